# Initial kernel scaffold; baseline (speedup 1.0000x reference)
import jax
import jax.numpy as jnp
from jax.experimental import pallas as pl


def kernel(x, edge_index, segment_ids, W1, W2, W3, wp1, bp1, wp2, bp2, wp3, bp3, Wd1, bd1, Wd2, bd2, Wd3, bd3):
    raise NotImplementedError("write your pallas kernel here")


# speedup vs baseline: 8.2662x; 8.2662x over previous
import functools

import jax
import jax.numpy as jnp
from jax import lax
from jax.experimental import pallas as pl
from jax.experimental.pallas import tpu as pltpu
from jax.experimental.pallas import tpu_sc as plsc

_N = 10000
_E = 320000
_G = 256
_NC = 2
_NS = 16
_CH = 128
_NCHUNK = _E // _CH
_RPT = 624
_TAIL = _N - _NS * _RPT
_ZR = 208
_DW = 128
_BN = 1000
_PREC = lax.Precision.HIGHEST

_mesh = plsc.VectorSubcoreMesh(core_axis_name="c", subcore_axis_name="s",
                               num_cores=_NC, num_subcores=_NS)


def _deg_body(dst_hbm, ones_hbm, zeros_hbm, out_hbm, dstv, ones_v, zrow_v,
              acc):
    c = lax.axis_index("c")
    s = lax.axis_index("s")
    wid = c * _NS + s
    pltpu.sync_copy(ones_hbm, ones_v)
    pltpu.sync_copy(zeros_hbm, zrow_v)
    for j in range(_RPT // _ZR):
        pltpu.sync_copy(zrow_v, acc.at[pl.ds(s * _RPT + j * _ZR, _ZR)])

    @pl.when(s == _NS - 1)
    def _():
        pltpu.sync_copy(zrow_v.at[pl.ds(0, _TAIL)],
                        acc.at[pl.ds(_NS * _RPT, _TAIL)])

    plsc.subcore_barrier()

    nworkers = _NC * _NS
    base, rem = _NCHUNK // nworkers, _NCHUNK % nworkers

    def do_chunk(chunk):
        e0 = chunk * _CH
        pltpu.sync_copy(dst_hbm.at[pl.ds(e0, _CH)], dstv)
        pltpu.sync_copy(ones_v, acc.at[dstv], add=True)

    def body(k, carry):
        do_chunk(wid + k * nworkers)
        return carry

    lax.fori_loop(0, base, body, 0)

    @pl.when(wid < rem)
    def _():
        do_chunk(wid + base * nworkers)

    plsc.subcore_barrier()
    pltpu.sync_copy(acc.at[pl.ds(s * _RPT, _RPT)],
                    out_hbm.at[c, pl.ds(s * _RPT, _RPT)])

    @pl.when(s == _NS - 1)
    def _():
        pltpu.sync_copy(acc.at[pl.ds(_NS * _RPT, _TAIL)],
                        out_hbm.at[c, pl.ds(_NS * _RPT, _TAIL)])


_deg_call = pl.kernel(
    _deg_body,
    out_type=jax.ShapeDtypeStruct((_NC, _N, _DW), jnp.float32),
    mesh=_mesh,
    scratch_types=[
        pltpu.VMEM((_CH,), jnp.int32),
        pltpu.VMEM((_CH, _DW), jnp.float32),
        pltpu.VMEM((_ZR, _DW), jnp.float32),
        pltpu.VMEM_SHARED((_N, _DW), jnp.float32),
    ],
)


def _make_edge_scatter(h, feature_split):
    nworkers = _NS if feature_split else _NC * _NS

    def body(table, srcm, dst_hbm, zeros_hbm, out_hbm, srcv, dstv, rows_v,
             zrow_v, acc, sem):
        c = lax.axis_index("c")
        s = lax.axis_index("s")
        wid = s if feature_split else c * _NS + s
        irow = c if feature_split else 0
        pltpu.sync_copy(zeros_hbm, zrow_v)
        for j in range(_RPT // _ZR):
            pltpu.sync_copy(zrow_v, acc.at[pl.ds(s * _RPT + j * _ZR, _ZR)])

        @pl.when(s == _NS - 1)
        def _():
            pltpu.sync_copy(zrow_v.at[pl.ds(0, _TAIL)],
                            acc.at[pl.ds(_NS * _RPT, _TAIL)])

        plsc.subcore_barrier()

        base, rem = _NCHUNK // nworkers, _NCHUNK % nworkers

        def do_chunk(chunk):
            e0 = chunk * _CH
            pltpu.sync_copy(srcm.at[irow, pl.ds(e0, _CH)], srcv)
            pltpu.sync_copy(dst_hbm.at[pl.ds(e0, _CH)], dstv)
            pltpu.async_copy(table.at[srcv], rows_v, sem).wait()
            pltpu.sync_copy(rows_v, acc.at[dstv], add=True)

        def body_k(k, carry):
            do_chunk(wid + k * nworkers)
            return carry

        lax.fori_loop(0, base, body_k, 0)

        @pl.when(wid < rem)
        def _():
            do_chunk(wid + base * nworkers)

        plsc.subcore_barrier()
        pltpu.sync_copy(acc.at[pl.ds(s * _RPT, _RPT)],
                        out_hbm.at[c, pl.ds(s * _RPT, _RPT)])

        @pl.when(s == _NS - 1)
        def _():
            pltpu.sync_copy(acc.at[pl.ds(_NS * _RPT, _TAIL)],
                            out_hbm.at[c, pl.ds(_NS * _RPT, _TAIL)])

    return pl.kernel(
        body,
        out_type=jax.ShapeDtypeStruct((_NC, _N, h), jnp.float32),
        mesh=_mesh,
        scratch_types=[
            pltpu.VMEM((_CH,), jnp.int32),
            pltpu.VMEM((_CH,), jnp.int32),
            pltpu.VMEM((_CH, h), jnp.float32),
            pltpu.VMEM((_ZR, h), jnp.float32),
            pltpu.VMEM_SHARED((_N, h), jnp.float32),
            pltpu.SemaphoreType.DMA,
        ],
    )


_scat_f128 = _make_edge_scatter(128, feature_split=True)
_scat_e128 = _make_edge_scatter(128, feature_split=False)


def _pre_body(deg_ref, x_ref, w1_ref, hw_ref, hn_ref):
    deg = deg_ref[0, :, 0:1] + deg_ref[1, :, 0:1] + 1.0
    norm = lax.rsqrt(deg)
    hw = jnp.dot(x_ref[...], w1_ref[...], preferred_element_type=jnp.float32,
                 precision=_PREC)
    hn = hw * norm
    hw_ref[...] = hw
    hn_ref[0] = hn[:, :128]
    hn_ref[1] = hn[:, 128:]


_pre_call = pl.pallas_call(
    _pre_body,
    grid=(_N // _BN,),
    in_specs=[
        pl.BlockSpec((_NC, _BN, _DW), lambda i: (0, i, 0)),
        pl.BlockSpec((_BN, 128), lambda i: (i, 0)),
        pl.BlockSpec((128, 256), lambda i: (0, 0)),
    ],
    out_specs=[
        pl.BlockSpec((_BN, 256), lambda i: (i, 0)),
        pl.BlockSpec((2, _BN, 128), lambda i: (0, i, 0)),
    ],
    out_shape=[
        jax.ShapeDtypeStruct((_N, 256), jnp.float32),
        jax.ShapeDtypeStruct((2, _N, 128), jnp.float32),
    ],
)


def _make_layer(h_in, h_next, feature_cat):
    def body(deg_ref, agg_ref, hw_ref, seg_ref, wp_ref, bp_ref, wn_ref,
             g_ref, hwn_ref, hnn_ref):
        i = pl.program_id(0)
        deg = deg_ref[0, :, 0:1] + deg_ref[1, :, 0:1] + 1.0
        norm = lax.rsqrt(deg)
        nsq = 1.0 / deg
        hw = hw_ref[...]
        if feature_cat:
            aggc = jnp.concatenate([agg_ref[0], agg_ref[1]], axis=1)
        else:
            aggc = agg_ref[0] + agg_ref[1]
        h = jnp.maximum(aggc * norm + hw * nsq, 0.0)
        att = jax.nn.sigmoid(
            jnp.dot(h, wp_ref[...], preferred_element_type=jnp.float32,
                    precision=_PREC) + bp_ref[0:1, 0:1])
        p = h * att
        seg = seg_ref[0]
        iota = lax.broadcasted_iota(jnp.int32, (_G, _BN), 0)
        oh_t = jnp.where(seg == iota, 1.0, 0.0)
        gpart = jnp.dot(oh_t, p, preferred_element_type=jnp.float32,
                        precision=_PREC)

        @pl.when(i == 0)
        def _():
            g_ref[...] = jnp.zeros_like(g_ref)

        g_ref[...] += gpart
        hwn = jnp.dot(h, wn_ref[...], preferred_element_type=jnp.float32,
                      precision=_PREC)
        hwn_ref[...] = hwn
        hnn = hwn * norm
        if h_next < 128:
            hnn = jnp.concatenate(
                [hnn, jnp.zeros((_BN, 128 - h_next), jnp.float32)], axis=1)
        hnn_ref[...] = hnn

    return pl.pallas_call(
        body,
        grid=(_N // _BN,),
        in_specs=[
            pl.BlockSpec((_NC, _BN, _DW), lambda i: (0, i, 0)),
            pl.BlockSpec((2, _BN, 128), lambda i: (0, i, 0)),
            pl.BlockSpec((_BN, h_in), lambda i: (i, 0)),
            pl.BlockSpec((1, 1, _BN), lambda i: (i, 0, 0)),
            pl.BlockSpec((h_in, 1), lambda i: (0, 0)),
            pl.BlockSpec((1, 128), lambda i: (0, 0)),
            pl.BlockSpec((h_in, h_next), lambda i: (0, 0)),
        ],
        out_specs=[
            pl.BlockSpec((_G, h_in), lambda i: (0, 0)),
            pl.BlockSpec((_BN, h_next), lambda i: (i, 0)),
            pl.BlockSpec((_BN, 128), lambda i: (i, 0)),
        ],
        out_shape=[
            jax.ShapeDtypeStruct((_G, h_in), jnp.float32),
            jax.ShapeDtypeStruct((_N, h_next), jnp.float32),
            jax.ShapeDtypeStruct((_N, 128), jnp.float32),
        ],
    )


_layer1 = _make_layer(256, 128, feature_cat=True)
_layer2 = _make_layer(128, 64, feature_cat=False)


def _final_body(deg_ref, agg_ref, hw_ref, seg_ref, wp_ref, bp_ref,
                g1_ref, g2_ref, wd1_ref, bd1_ref, wd2_ref, bd2_ref,
                wd3_ref, bd3_ref, out_ref, g3_acc):
    i = pl.program_id(0)
    deg = deg_ref[0, :, 0:1] + deg_ref[1, :, 0:1] + 1.0
    norm = lax.rsqrt(deg)
    nsq = 1.0 / deg
    aggc = agg_ref[0, :, 0:64] + agg_ref[1, :, 0:64]
    h = jnp.maximum(aggc * norm + hw_ref[...] * nsq, 0.0)
    att = jax.nn.sigmoid(
        jnp.dot(h, wp_ref[...], preferred_element_type=jnp.float32,
                precision=_PREC) + bp_ref[0:1, 0:1])
    p = h * att
    seg = seg_ref[0]
    iota = lax.broadcasted_iota(jnp.int32, (_G, _BN), 0)
    oh_t = jnp.where(seg == iota, 1.0, 0.0)
    gpart = jnp.dot(oh_t, p, preferred_element_type=jnp.float32,
                    precision=_PREC)

    @pl.when(i == 0)
    def _():
        g3_acc[...] = jnp.zeros_like(g3_acc)

    g3_acc[...] += gpart

    @pl.when(i == _N // _BN - 1)
    def _():
        g1 = g1_ref[...]
        g2 = g2_ref[...]
        g3 = g3_acc[...]
        z1 = jnp.dot(g1, wd1_ref[0:256, :], preferred_element_type=jnp.float32,
                     precision=_PREC)
        z1 += jnp.dot(g2, wd1_ref[256:384, :],
                      preferred_element_type=jnp.float32, precision=_PREC)
        z1 += jnp.dot(g3, wd1_ref[384:448, :],
                      preferred_element_type=jnp.float32, precision=_PREC)
        z1 = jnp.maximum(z1 + bd1_ref[...], 0.0)
        z2 = jnp.maximum(
            jnp.dot(z1, wd2_ref[...], preferred_element_type=jnp.float32,
                    precision=_PREC) + bd2_ref[...], 0.0)
        out_ref[...] = jnp.dot(
            z2, wd3_ref[...], preferred_element_type=jnp.float32,
            precision=_PREC) + bd3_ref[...]


_final_call = pl.pallas_call(
    _final_body,
    grid=(_N // _BN,),
    in_specs=[
        pl.BlockSpec((_NC, _BN, _DW), lambda i: (0, i, 0)),
        pl.BlockSpec((2, _BN, 128), lambda i: (0, i, 0)),
        pl.BlockSpec((_BN, 64), lambda i: (i, 0)),
        pl.BlockSpec((1, 1, _BN), lambda i: (i, 0, 0)),
        pl.BlockSpec((64, 1), lambda i: (0, 0)),
        pl.BlockSpec((1, 128), lambda i: (0, 0)),
        pl.BlockSpec((_G, 256), lambda i: (0, 0)),
        pl.BlockSpec((_G, 128), lambda i: (0, 0)),
        pl.BlockSpec((448, 256), lambda i: (0, 0)),
        pl.BlockSpec((1, 256), lambda i: (0, 0)),
        pl.BlockSpec((256, 128), lambda i: (0, 0)),
        pl.BlockSpec((1, 128), lambda i: (0, 0)),
        pl.BlockSpec((128, 32), lambda i: (0, 0)),
        pl.BlockSpec((1, 32), lambda i: (0, 0)),
    ],
    out_specs=pl.BlockSpec((_G, 32), lambda i: (0, 0)),
    out_shape=jax.ShapeDtypeStruct((_G, 32), jnp.float32),
    scratch_shapes=[pltpu.VMEM((_G, 64), jnp.float32)],
)


def kernel(x, edge_index, segment_ids, W1, W2, W3, wp1, bp1, wp2, bp2,
           wp3, bp3, Wd1, bd1, Wd2, bd2, Wd3, bd3):
    src = edge_index[0]
    dst = edge_index[1]
    seg3 = segment_ids.reshape(_N // _BN, 1, _BN)
    src_f = jnp.stack([src, src + _N])
    src_e = src.reshape(1, _E)

    def b128(b):
        return jnp.broadcast_to(b.reshape(1, 1), (1, 128))

    z128 = jnp.zeros((_ZR, 128), jnp.float32)
    deg2 = _deg_call(dst, jnp.ones((_CH, _DW), jnp.float32),
                     jnp.zeros((_ZR, _DW), jnp.float32))
    hw1, hn1 = _pre_call(deg2, x, W1)
    agg1 = _scat_f128(hn1.reshape(2 * _N, 128), src_f, dst, z128)
    g1, hw2, hn2 = _layer1(deg2, agg1, hw1, seg3, wp1, b128(bp1), W2)
    agg2 = _scat_e128(hn2, src_e, dst, z128)
    g2, hw3, hn3 = _layer2(deg2, agg2, hw2, seg3, wp2, b128(bp2), W3)
    agg3 = _scat_e128(hn3, src_e, dst, z128)
    out = _final_call(deg2, agg3, hw3, seg3, wp3, b128(bp3), g1, g2,
                      Wd1, bd1.reshape(1, 256), Wd2, bd2.reshape(1, 128),
                      Wd3, bd3.reshape(1, 32))
    return out


# speedup vs baseline: 13.5727x; 1.6420x over previous
import functools

import jax
import jax.numpy as jnp
from jax import lax
from jax.experimental import pallas as pl
from jax.experimental.pallas import tpu as pltpu
from jax.experimental.pallas import tpu_sc as plsc

_N = 10000
_E = 320000
_G = 256
_NC = 2
_NS = 16
_CH = 128
_NCHUNK = _E // _CH
_NCHPAD = _NCHUNK + 4
_RPT = 624
_TAIL = _N - _NS * _RPT
_ZR = 208
_DW = 128
_BN = 1000
_PREC = lax.Precision.HIGHEST

_mesh = plsc.VectorSubcoreMesh(core_axis_name="c", subcore_axis_name="s",
                               num_cores=_NC, num_subcores=_NS)


def _deg_body(dst_hbm, ones_hbm, zeros_hbm, out_hbm, dstv, ones_v, zrow_v,
              acc):
    c = lax.axis_index("c")
    s = lax.axis_index("s")
    wid = c * _NS + s
    pltpu.sync_copy(ones_hbm, ones_v)
    pltpu.sync_copy(zeros_hbm, zrow_v)
    for j in range(_RPT // _ZR):
        pltpu.sync_copy(zrow_v, acc.at[pl.ds(s * _RPT + j * _ZR, _ZR)])

    @pl.when(s == _NS - 1)
    def _():
        pltpu.sync_copy(zrow_v.at[pl.ds(0, _TAIL)],
                        acc.at[pl.ds(_NS * _RPT, _TAIL)])

    plsc.subcore_barrier()

    nworkers = _NC * _NS
    base, rem = _NCHUNK // nworkers, _NCHUNK % nworkers

    def do_chunk(chunk):
        e0 = chunk * _CH
        pltpu.sync_copy(dst_hbm.at[pl.ds(e0, _CH)], dstv)
        pltpu.sync_copy(ones_v, acc.at[dstv], add=True)

    def body(k, carry):
        do_chunk(wid + k * nworkers)
        return carry

    lax.fori_loop(0, base, body, 0)

    @pl.when(wid < rem)
    def _():
        do_chunk(wid + base * nworkers)

    plsc.subcore_barrier()
    pltpu.sync_copy(acc.at[pl.ds(s * _RPT, _RPT)],
                    out_hbm.at[c, pl.ds(s * _RPT, _RPT)])

    @pl.when(s == _NS - 1)
    def _():
        pltpu.sync_copy(acc.at[pl.ds(_NS * _RPT, _TAIL)],
                        out_hbm.at[c, pl.ds(_NS * _RPT, _TAIL)])


_deg_call = pl.kernel(
    _deg_body,
    out_type=jax.ShapeDtypeStruct((_NC, _N, _DW), jnp.float32),
    mesh=_mesh,
    scratch_types=[
        pltpu.VMEM((_CH,), jnp.int32),
        pltpu.VMEM((_CH, _DW), jnp.float32),
        pltpu.VMEM((_ZR, _DW), jnp.float32),
        pltpu.VMEM_SHARED((_N, _DW), jnp.float32),
    ],
)


_PC = 24
_SLOT = 80
_PART = _NCHUNK - _SLOT * (_NC * _NS - 1)


def _make_edge_scatter(h):
    def body(table, srcm, dstm, zeros_hbm, out_hbm, srcb, dstb, rows0, rows1,
             acc, semg0, semg1, sems0, sems1):
        c = lax.axis_index("c")
        s = lax.axis_index("s")
        wid = c * _NS + s
        pltpu.sync_copy(zeros_hbm, rows0)
        for j in range(4):
            pltpu.sync_copy(rows0, acc.at[pl.ds(s * _RPT + j * _CH, _CH)])
        pltpu.sync_copy(rows0.at[pl.ds(0, _RPT - 4 * _CH)],
                        acc.at[pl.ds(s * _RPT + 4 * _CH, _RPT - 4 * _CH)])

        @pl.when(s == _NS - 1)
        def _():
            pltpu.sync_copy(rows0.at[pl.ds(0, _TAIL)],
                            acc.at[pl.ds(_NS * _RPT, _TAIL)])

        plsc.subcore_barrier()

        k0 = wid * _SLOT
        nch = jnp.minimum(_SLOT, _NCHUNK - k0)
        rows = (rows0, rows1)
        semg = (semg0, semg1)
        sems = (sems0, sems1)

        def gath(k, b):
            return pltpu.make_async_copy(table.at[srcb.at[k]], rows[b],
                                         semg[b])

        def scat(k, b):
            return pltpu.make_async_copy(rows[b], acc.at[dstb.at[k]],
                                         sems[b])

        def piece(p, carry):
            base = p * _PC
            pltpu.sync_copy(srcm.at[0, pl.ds(k0 + base, _PC)], srcb)
            pltpu.sync_copy(dstm.at[0, pl.ds(k0 + base, _PC)], dstb)
            m = jnp.minimum(_PC, nch - base)
            pltpu.async_copy(table.at[srcb.at[0]], rows0, semg0)

            def phase(k, b):
                @pl.when(k + 1 < m)
                def _():
                    @pl.when(k >= 1)
                    def _():
                        scat(k - 1, 1 - b).wait()
                    pltpu.async_copy(table.at[srcb.at[k + 1]], rows[1 - b],
                                     semg[1 - b])
                gath(k, b).wait()
                pltpu.async_copy(rows[b], acc.at[dstb.at[k]], sems[b],
                                 add=True)

            def body2(k2, carry2):
                k = 2 * k2
                phase(k, 0)

                @pl.when(k + 1 < m)
                def _():
                    phase(k + 1, 1)

                return carry2

            lax.fori_loop(0, (m + 1) // 2, body2, 0)

            par = m % 2

            @pl.when((m >= 2) & (par == 0))
            def _():
                scat(0, 0).wait()

            @pl.when((m >= 2) & (par == 1))
            def _():
                scat(0, 1).wait()

            @pl.when(par == 1)
            def _():
                scat(0, 0).wait()

            @pl.when(par == 0)
            def _():
                scat(0, 1).wait()

            return carry

        lax.fori_loop(0, (nch + _PC - 1) // _PC, piece, 0)

        plsc.subcore_barrier()
        pltpu.sync_copy(acc.at[pl.ds(s * _RPT, _RPT)],
                        out_hbm.at[c, pl.ds(s * _RPT, _RPT)])

        @pl.when(s == _NS - 1)
        def _():
            pltpu.sync_copy(acc.at[pl.ds(_NS * _RPT, _TAIL)],
                            out_hbm.at[c, pl.ds(_NS * _RPT, _TAIL)])

    return pl.kernel(
        body,
        out_type=jax.ShapeDtypeStruct((_NC, _N, h), jnp.float32),
        mesh=_mesh,
        scratch_types=[
            pltpu.VMEM((_PC, _CH), jnp.int32),
            pltpu.VMEM((_PC, _CH), jnp.int32),
            pltpu.VMEM((_CH, h), jnp.float32),
            pltpu.VMEM((_CH, h), jnp.float32),
            pltpu.VMEM_SHARED((_N, h), jnp.float32),
            pltpu.SemaphoreType.DMA,
            pltpu.SemaphoreType.DMA,
            pltpu.SemaphoreType.DMA,
            pltpu.SemaphoreType.DMA,
        ],
    )


_scat_e128 = _make_edge_scatter(128)


def _pre_body(deg_ref, x_ref, w1_ref, hw_ref, hn_ref):
    deg = deg_ref[0, :, 0:1] + deg_ref[1, :, 0:1] + 1.0
    norm = lax.rsqrt(deg)
    hw = jnp.dot(x_ref[...], w1_ref[...], preferred_element_type=jnp.float32,
                 precision=_PREC)
    hn = hw * norm
    hw_ref[...] = hw
    hn_ref[0] = hn[:, :128]
    hn_ref[1] = hn[:, 128:]


_pre_call = pl.pallas_call(
    _pre_body,
    grid=(_N // _BN,),
    in_specs=[
        pl.BlockSpec((_NC, _BN, _DW), lambda i: (0, i, 0)),
        pl.BlockSpec((_BN, 128), lambda i: (i, 0)),
        pl.BlockSpec((128, 256), lambda i: (0, 0)),
    ],
    out_specs=[
        pl.BlockSpec((_BN, 256), lambda i: (i, 0)),
        pl.BlockSpec((2, _BN, 128), lambda i: (0, i, 0)),
    ],
    out_shape=[
        jax.ShapeDtypeStruct((_N, 256), jnp.float32),
        jax.ShapeDtypeStruct((2, _N, 128), jnp.float32),
    ],
)


def _make_layer(h_in, h_next, two_aggs):
    def body(deg_ref, *refs):
        if two_aggs:
            agg_a, agg_b, hw_ref, seg_ref, wp_ref, bp_ref, wn_ref, \
                g_ref, hwn_ref, hnn_ref = refs
        else:
            agg_a, hw_ref, seg_ref, wp_ref, bp_ref, wn_ref, \
                g_ref, hwn_ref, hnn_ref = refs
        i = pl.program_id(0)
        deg = deg_ref[0, :, 0:1] + deg_ref[1, :, 0:1] + 1.0
        norm = lax.rsqrt(deg)
        nsq = 1.0 / deg
        hw = hw_ref[...]
        if two_aggs:
            aggc = jnp.concatenate([agg_a[0] + agg_a[1],
                                    agg_b[0] + agg_b[1]], axis=1)
        else:
            aggc = agg_a[0] + agg_a[1]
        h = jnp.maximum(aggc * norm + hw * nsq, 0.0)
        att = jax.nn.sigmoid(
            jnp.dot(h, wp_ref[...], preferred_element_type=jnp.float32,
                    precision=_PREC) + bp_ref[0:1, 0:1])
        p = h * att
        seg = seg_ref[0]
        iota = lax.broadcasted_iota(jnp.int32, (_G, _BN), 0)
        oh_t = jnp.where(seg == iota, 1.0, 0.0)
        gpart = jnp.dot(oh_t, p, preferred_element_type=jnp.float32,
                        precision=_PREC)

        @pl.when(i == 0)
        def _():
            g_ref[...] = jnp.zeros_like(g_ref)

        g_ref[...] += gpart
        hwn = jnp.dot(h, wn_ref[...], preferred_element_type=jnp.float32,
                      precision=_PREC)
        hwn_ref[...] = hwn
        hnn = hwn * norm
        if h_next < 128:
            hnn = jnp.concatenate(
                [hnn, jnp.zeros((_BN, 128 - h_next), jnp.float32)], axis=1)
        hnn_ref[...] = hnn

    agg_specs = [pl.BlockSpec((2, _BN, 128), lambda i: (0, i, 0))]
    if two_aggs:
        agg_specs.append(pl.BlockSpec((2, _BN, 128), lambda i: (0, i, 0)))
    return pl.pallas_call(
        body,
        grid=(_N // _BN,),
        in_specs=[
            pl.BlockSpec((_NC, _BN, _DW), lambda i: (0, i, 0)),
            *agg_specs,
            pl.BlockSpec((_BN, h_in), lambda i: (i, 0)),
            pl.BlockSpec((1, 1, _BN), lambda i: (i, 0, 0)),
            pl.BlockSpec((h_in, 1), lambda i: (0, 0)),
            pl.BlockSpec((1, 128), lambda i: (0, 0)),
            pl.BlockSpec((h_in, h_next), lambda i: (0, 0)),
        ],
        out_specs=[
            pl.BlockSpec((_G, h_in), lambda i: (0, 0)),
            pl.BlockSpec((_BN, h_next), lambda i: (i, 0)),
            pl.BlockSpec((_BN, 128), lambda i: (i, 0)),
        ],
        out_shape=[
            jax.ShapeDtypeStruct((_G, h_in), jnp.float32),
            jax.ShapeDtypeStruct((_N, h_next), jnp.float32),
            jax.ShapeDtypeStruct((_N, 128), jnp.float32),
        ],
    )


_layer1 = _make_layer(256, 128, two_aggs=True)
_layer2 = _make_layer(128, 64, two_aggs=False)


def _final_body(deg_ref, agg_ref, hw_ref, seg_ref, wp_ref, bp_ref,
                g1_ref, g2_ref, wd1_ref, bd1_ref, wd2_ref, bd2_ref,
                wd3_ref, bd3_ref, out_ref, g3_acc):
    i = pl.program_id(0)
    deg = deg_ref[0, :, 0:1] + deg_ref[1, :, 0:1] + 1.0
    norm = lax.rsqrt(deg)
    nsq = 1.0 / deg
    aggc = agg_ref[0, :, 0:64] + agg_ref[1, :, 0:64]
    h = jnp.maximum(aggc * norm + hw_ref[...] * nsq, 0.0)
    att = jax.nn.sigmoid(
        jnp.dot(h, wp_ref[...], preferred_element_type=jnp.float32,
                precision=_PREC) + bp_ref[0:1, 0:1])
    p = h * att
    seg = seg_ref[0]
    iota = lax.broadcasted_iota(jnp.int32, (_G, _BN), 0)
    oh_t = jnp.where(seg == iota, 1.0, 0.0)
    gpart = jnp.dot(oh_t, p, preferred_element_type=jnp.float32,
                    precision=_PREC)

    @pl.when(i == 0)
    def _():
        g3_acc[...] = jnp.zeros_like(g3_acc)

    g3_acc[...] += gpart

    @pl.when(i == _N // _BN - 1)
    def _():
        g1 = g1_ref[...]
        g2 = g2_ref[...]
        g3 = g3_acc[...]
        z1 = jnp.dot(g1, wd1_ref[0:256, :], preferred_element_type=jnp.float32,
                     precision=_PREC)
        z1 += jnp.dot(g2, wd1_ref[256:384, :],
                      preferred_element_type=jnp.float32, precision=_PREC)
        z1 += jnp.dot(g3, wd1_ref[384:448, :],
                      preferred_element_type=jnp.float32, precision=_PREC)
        z1 = jnp.maximum(z1 + bd1_ref[...], 0.0)
        z2 = jnp.maximum(
            jnp.dot(z1, wd2_ref[...], preferred_element_type=jnp.float32,
                    precision=_PREC) + bd2_ref[...], 0.0)
        out_ref[...] = jnp.dot(
            z2, wd3_ref[...], preferred_element_type=jnp.float32,
            precision=_PREC) + bd3_ref[...]


_final_call = pl.pallas_call(
    _final_body,
    grid=(_N // _BN,),
    in_specs=[
        pl.BlockSpec((_NC, _BN, _DW), lambda i: (0, i, 0)),
        pl.BlockSpec((2, _BN, 128), lambda i: (0, i, 0)),
        pl.BlockSpec((_BN, 64), lambda i: (i, 0)),
        pl.BlockSpec((1, 1, _BN), lambda i: (i, 0, 0)),
        pl.BlockSpec((64, 1), lambda i: (0, 0)),
        pl.BlockSpec((1, 128), lambda i: (0, 0)),
        pl.BlockSpec((_G, 256), lambda i: (0, 0)),
        pl.BlockSpec((_G, 128), lambda i: (0, 0)),
        pl.BlockSpec((448, 256), lambda i: (0, 0)),
        pl.BlockSpec((1, 256), lambda i: (0, 0)),
        pl.BlockSpec((256, 128), lambda i: (0, 0)),
        pl.BlockSpec((1, 128), lambda i: (0, 0)),
        pl.BlockSpec((128, 32), lambda i: (0, 0)),
        pl.BlockSpec((1, 32), lambda i: (0, 0)),
    ],
    out_specs=pl.BlockSpec((_G, 32), lambda i: (0, 0)),
    out_shape=jax.ShapeDtypeStruct((_G, 32), jnp.float32),
    scratch_shapes=[pltpu.VMEM((_G, 64), jnp.float32)],
)


def kernel(x, edge_index, segment_ids, W1, W2, W3, wp1, bp1, wp2, bp2,
           wp3, bp3, Wd1, bd1, Wd2, bd2, Wd3, bd3):
    src = edge_index[0]
    dst = edge_index[1]
    seg3 = segment_ids.reshape(_N // _BN, 1, _BN)
    def chunked(a):
        a3 = a.reshape(1, _NCHUNK, _CH)
        return jnp.pad(a3, ((0, 0), (0, _NCHPAD - _NCHUNK), (0, 0)))

    src_lo = chunked(src)
    src_hi = chunked(src + _N)
    dst_c = chunked(dst)

    def b128(b):
        return jnp.broadcast_to(b.reshape(1, 1), (1, 128))

    z128 = jnp.zeros((_CH, 128), jnp.float32)
    deg2 = _deg_call(dst, jnp.ones((_CH, _DW), jnp.float32),
                     jnp.zeros((_ZR, _DW), jnp.float32))
    hw1, hn1 = _pre_call(deg2, x, W1)
    t1 = hn1.reshape(2 * _N, 128)
    agg1a = _scat_e128(t1, src_lo, dst_c, z128)
    agg1b = _scat_e128(t1, src_hi, dst_c, z128)
    g1, hw2, hn2 = _layer1(deg2, agg1a, agg1b, hw1, seg3, wp1, b128(bp1), W2)
    agg2 = _scat_e128(hn2, src_lo, dst_c, z128)
    g2, hw3, hn3 = _layer2(deg2, agg2, hw2, seg3, wp2, b128(bp2), W3)
    agg3 = _scat_e128(hn3, src_lo, dst_c, z128)
    out = _final_call(deg2, agg3, hw3, seg3, wp3, b128(bp3), g1, g2,
                      Wd1, bd1.reshape(1, 256), Wd2, bd2.reshape(1, 128),
                      Wd3, bd3.reshape(1, 32))
    return out


# speedup vs baseline: 14.1029x; 1.0391x over previous
import functools

import jax
import jax.numpy as jnp
from jax import lax
from jax.experimental import pallas as pl
from jax.experimental.pallas import tpu as pltpu
from jax.experimental.pallas import tpu_sc as plsc

_N = 10000
_E = 320000
_G = 256
_NC = 2
_NS = 16
_CH = 128
_NCHUNK = _E // _CH
_NCHPAD = _NCHUNK + 4
_RPT = 624
_TAIL = _N - _NS * _RPT
_ZR = 208
_DW = 128
_BN = 1000
_PREC = lax.Precision.HIGHEST
_PC = 24
_SLOT = 80

_mesh = plsc.VectorSubcoreMesh(core_axis_name="c", subcore_axis_name="s",
                               num_cores=_NC, num_subcores=_NS)


def _deg_body(dstm, ones_hbm, zeros_hbm, out_hbm, dstb, ones_v, zrow_v,
              acc, sems):
    c = lax.axis_index("c")
    s = lax.axis_index("s")
    wid = c * _NS + s
    pltpu.sync_copy(ones_hbm, ones_v)
    pltpu.sync_copy(zeros_hbm, zrow_v)
    for j in range(_RPT // _ZR):
        pltpu.sync_copy(zrow_v, acc.at[pl.ds(s * _RPT + j * _ZR, _ZR)])

    @pl.when(s == _NS - 1)
    def _():
        pltpu.sync_copy(zrow_v.at[pl.ds(0, _TAIL)],
                        acc.at[pl.ds(_NS * _RPT, _TAIL)])

    plsc.subcore_barrier()

    k0 = wid * _SLOT
    nch = jnp.minimum(_SLOT, _NCHUNK - k0)

    def piece(p, carry):
        base = p * _PC
        pltpu.sync_copy(dstm.at[0, pl.ds(k0 + base, _PC)], dstb)
        m = jnp.minimum(_PC, nch - base)

        def fire(k, carry2):
            pltpu.async_copy(ones_v, acc.at[dstb.at[k]], sems, add=True)
            return carry2

        lax.fori_loop(0, m, fire, 0)

        def drain(k, carry2):
            pltpu.make_async_copy(ones_v, acc.at[dstb.at[0]], sems).wait()
            return carry2

        lax.fori_loop(0, m, drain, 0)
        return carry

    lax.fori_loop(0, (nch + _PC - 1) // _PC, piece, 0)

    plsc.subcore_barrier()
    pltpu.sync_copy(acc.at[pl.ds(s * _RPT, _RPT)],
                    out_hbm.at[c, pl.ds(s * _RPT, _RPT)])

    @pl.when(s == _NS - 1)
    def _():
        pltpu.sync_copy(acc.at[pl.ds(_NS * _RPT, _TAIL)],
                        out_hbm.at[c, pl.ds(_NS * _RPT, _TAIL)])


_deg_call = pl.kernel(
    _deg_body,
    out_type=jax.ShapeDtypeStruct((_NC, _N, _DW), jnp.float32),
    mesh=_mesh,
    scratch_types=[
        pltpu.VMEM((_PC, _CH), jnp.int32),
        pltpu.VMEM((_CH, _DW), jnp.float32),
        pltpu.VMEM((_ZR, _DW), jnp.float32),
        pltpu.VMEM_SHARED((_N, _DW), jnp.float32),
        pltpu.SemaphoreType.DMA,
    ],
)


def _make_edge_scatter(h):
    def body(table, srcm, dstm, zeros_hbm, out_hbm, srcb, dstb, rows0, rows1,
             acc, semg0, semg1, sems0, sems1):
        c = lax.axis_index("c")
        s = lax.axis_index("s")
        wid = c * _NS + s
        pltpu.sync_copy(zeros_hbm, rows0)
        for j in range(4):
            pltpu.sync_copy(rows0, acc.at[pl.ds(s * _RPT + j * _CH, _CH)])
        pltpu.sync_copy(rows0.at[pl.ds(0, _RPT - 4 * _CH)],
                        acc.at[pl.ds(s * _RPT + 4 * _CH, _RPT - 4 * _CH)])

        @pl.when(s == _NS - 1)
        def _():
            pltpu.sync_copy(rows0.at[pl.ds(0, _TAIL)],
                            acc.at[pl.ds(_NS * _RPT, _TAIL)])

        plsc.subcore_barrier()

        k0 = wid * _SLOT
        nch = jnp.minimum(_SLOT, _NCHUNK - k0)
        rows = (rows0, rows1)
        semg = (semg0, semg1)
        sems = (sems0, sems1)

        def gath(k, b):
            return pltpu.make_async_copy(table.at[srcb.at[k]], rows[b],
                                         semg[b])

        def scat(k, b):
            return pltpu.make_async_copy(rows[b], acc.at[dstb.at[k]],
                                         sems[b])

        def piece(p, carry):
            base = p * _PC
            pltpu.sync_copy(srcm.at[0, pl.ds(k0 + base, _PC)], srcb)
            pltpu.sync_copy(dstm.at[0, pl.ds(k0 + base, _PC)], dstb)
            m = jnp.minimum(_PC, nch - base)
            pltpu.async_copy(table.at[srcb.at[0]], rows0, semg0)

            def phase(k, b):
                @pl.when(k + 1 < m)
                def _():
                    @pl.when(k >= 1)
                    def _():
                        scat(k - 1, 1 - b).wait()
                    pltpu.async_copy(table.at[srcb.at[k + 1]], rows[1 - b],
                                     semg[1 - b])
                gath(k, b).wait()
                pltpu.async_copy(rows[b], acc.at[dstb.at[k]], sems[b],
                                 add=True)

            def body2(k2, carry2):
                k = 2 * k2
                phase(k, 0)

                @pl.when(k + 1 < m)
                def _():
                    phase(k + 1, 1)

                return carry2

            lax.fori_loop(0, (m + 1) // 2, body2, 0)

            par = m % 2

            @pl.when((m >= 2) & (par == 0))
            def _():
                scat(0, 0).wait()

            @pl.when((m >= 2) & (par == 1))
            def _():
                scat(0, 1).wait()

            @pl.when(par == 1)
            def _():
                scat(0, 0).wait()

            @pl.when(par == 0)
            def _():
                scat(0, 1).wait()

            return carry

        lax.fori_loop(0, (nch + _PC - 1) // _PC, piece, 0)

        plsc.subcore_barrier()
        pltpu.sync_copy(acc.at[pl.ds(s * _RPT, _RPT)],
                        out_hbm.at[c, pl.ds(s * _RPT, _RPT)])

        @pl.when(s == _NS - 1)
        def _():
            pltpu.sync_copy(acc.at[pl.ds(_NS * _RPT, _TAIL)],
                            out_hbm.at[c, pl.ds(_NS * _RPT, _TAIL)])

    return pl.kernel(
        body,
        out_type=jax.ShapeDtypeStruct((_NC, _N, h), jnp.float32),
        mesh=_mesh,
        scratch_types=[
            pltpu.VMEM((_PC, _CH), jnp.int32),
            pltpu.VMEM((_PC, _CH), jnp.int32),
            pltpu.VMEM((_CH, h), jnp.float32),
            pltpu.VMEM((_CH, h), jnp.float32),
            pltpu.VMEM_SHARED((_N, h), jnp.float32),
            pltpu.SemaphoreType.DMA,
            pltpu.SemaphoreType.DMA,
            pltpu.SemaphoreType.DMA,
            pltpu.SemaphoreType.DMA,
        ],
    )


_scat_e128 = _make_edge_scatter(128)


def _pre_body(deg_ref, x_ref, w1_ref, hw_ref, hn_ref):
    deg = deg_ref[0, :, 0:1] + deg_ref[1, :, 0:1] + 1.0
    norm = lax.rsqrt(deg)
    hw = jnp.dot(x_ref[...], w1_ref[...], preferred_element_type=jnp.float32,
                 precision=_PREC)
    hn = hw * norm
    hw_ref[...] = hw
    hn_ref[0] = hn[:, :128]
    hn_ref[1] = hn[:, 128:]


_pre_call = pl.pallas_call(
    _pre_body,
    grid=(_N // _BN,),
    in_specs=[
        pl.BlockSpec((_NC, _BN, _DW), lambda i: (0, i, 0)),
        pl.BlockSpec((_BN, 128), lambda i: (i, 0)),
        pl.BlockSpec((128, 256), lambda i: (0, 0)),
    ],
    out_specs=[
        pl.BlockSpec((_BN, 256), lambda i: (i, 0)),
        pl.BlockSpec((2, _BN, 128), lambda i: (0, i, 0)),
    ],
    out_shape=[
        jax.ShapeDtypeStruct((_N, 256), jnp.float32),
        jax.ShapeDtypeStruct((2, _N, 128), jnp.float32),
    ],
)


def _make_layer(h_in, h_next, two_aggs):
    def body(deg_ref, *refs):
        if two_aggs:
            agg_a, agg_b, hw_ref, seg_ref, wp_ref, bp_ref, wn_ref, \
                g_ref, hwn_ref, hnn_ref = refs
        else:
            agg_a, hw_ref, seg_ref, wp_ref, bp_ref, wn_ref, \
                g_ref, hwn_ref, hnn_ref = refs
        i = pl.program_id(0)
        deg = deg_ref[0, :, 0:1] + deg_ref[1, :, 0:1] + 1.0
        norm = lax.rsqrt(deg)
        nsq = 1.0 / deg
        hw = hw_ref[...]
        if two_aggs:
            aggc = jnp.concatenate([agg_a[0] + agg_a[1],
                                    agg_b[0] + agg_b[1]], axis=1)
        else:
            aggc = agg_a[0] + agg_a[1]
        h = jnp.maximum(aggc * norm + hw * nsq, 0.0)
        att = jax.nn.sigmoid(
            jnp.dot(h, wp_ref[...], preferred_element_type=jnp.float32,
                    precision=_PREC) + bp_ref[0:1, 0:1])
        p = h * att
        seg = seg_ref[0]
        iota = lax.broadcasted_iota(jnp.int32, (_G, _BN), 0)
        oh_t = jnp.where(seg == iota, 1.0, 0.0)
        gpart = jnp.dot(oh_t, p, preferred_element_type=jnp.float32,
                        precision=_PREC)

        @pl.when(i == 0)
        def _():
            g_ref[...] = jnp.zeros_like(g_ref)

        g_ref[...] += gpart
        hwn = jnp.dot(h, wn_ref[...], preferred_element_type=jnp.float32,
                      precision=_PREC)
        hwn_ref[...] = hwn
        hnn = hwn * norm
        if h_next < 128:
            hnn = jnp.concatenate(
                [hnn, jnp.zeros((_BN, 128 - h_next), jnp.float32)], axis=1)
        hnn_ref[...] = hnn

    agg_specs = [pl.BlockSpec((2, _BN, 128), lambda i: (0, i, 0))]
    if two_aggs:
        agg_specs.append(pl.BlockSpec((2, _BN, 128), lambda i: (0, i, 0)))
    return pl.pallas_call(
        body,
        grid=(_N // _BN,),
        in_specs=[
            pl.BlockSpec((_NC, _BN, _DW), lambda i: (0, i, 0)),
            *agg_specs,
            pl.BlockSpec((_BN, h_in), lambda i: (i, 0)),
            pl.BlockSpec((1, 1, _BN), lambda i: (i, 0, 0)),
            pl.BlockSpec((h_in, 1), lambda i: (0, 0)),
            pl.BlockSpec((1, 128), lambda i: (0, 0)),
            pl.BlockSpec((h_in, h_next), lambda i: (0, 0)),
        ],
        out_specs=[
            pl.BlockSpec((_G, h_in), lambda i: (0, 0)),
            pl.BlockSpec((_BN, h_next), lambda i: (i, 0)),
            pl.BlockSpec((_BN, 128), lambda i: (i, 0)),
        ],
        out_shape=[
            jax.ShapeDtypeStruct((_G, h_in), jnp.float32),
            jax.ShapeDtypeStruct((_N, h_next), jnp.float32),
            jax.ShapeDtypeStruct((_N, 128), jnp.float32),
        ],
    )


_layer1 = _make_layer(256, 128, two_aggs=True)
_layer2 = _make_layer(128, 64, two_aggs=False)


def _final_body(deg_ref, agg_ref, hw_ref, seg_ref, wp_ref, bp_ref,
                g1_ref, g2_ref, wd1_ref, bd1_ref, wd2_ref, bd2_ref,
                wd3_ref, bd3_ref, out_ref, g3_acc):
    i = pl.program_id(0)
    deg = deg_ref[0, :, 0:1] + deg_ref[1, :, 0:1] + 1.0
    norm = lax.rsqrt(deg)
    nsq = 1.0 / deg
    aggc = agg_ref[0, :, 0:64] + agg_ref[1, :, 0:64]
    h = jnp.maximum(aggc * norm + hw_ref[...] * nsq, 0.0)
    att = jax.nn.sigmoid(
        jnp.dot(h, wp_ref[...], preferred_element_type=jnp.float32,
                precision=_PREC) + bp_ref[0:1, 0:1])
    p = h * att
    seg = seg_ref[0]
    iota = lax.broadcasted_iota(jnp.int32, (_G, _BN), 0)
    oh_t = jnp.where(seg == iota, 1.0, 0.0)
    gpart = jnp.dot(oh_t, p, preferred_element_type=jnp.float32,
                    precision=_PREC)

    @pl.when(i == 0)
    def _():
        g3_acc[...] = jnp.zeros_like(g3_acc)

    g3_acc[...] += gpart

    @pl.when(i == _N // _BN - 1)
    def _():
        g1 = g1_ref[...]
        g2 = g2_ref[...]
        g3 = g3_acc[...]
        z1 = jnp.dot(g1, wd1_ref[0:256, :], preferred_element_type=jnp.float32,
                     precision=_PREC)
        z1 += jnp.dot(g2, wd1_ref[256:384, :],
                      preferred_element_type=jnp.float32, precision=_PREC)
        z1 += jnp.dot(g3, wd1_ref[384:448, :],
                      preferred_element_type=jnp.float32, precision=_PREC)
        z1 = jnp.maximum(z1 + bd1_ref[...], 0.0)
        z2 = jnp.maximum(
            jnp.dot(z1, wd2_ref[...], preferred_element_type=jnp.float32,
                    precision=_PREC) + bd2_ref[...], 0.0)
        out_ref[...] = jnp.dot(
            z2, wd3_ref[...], preferred_element_type=jnp.float32,
            precision=_PREC) + bd3_ref[...]


_final_call = pl.pallas_call(
    _final_body,
    grid=(_N // _BN,),
    in_specs=[
        pl.BlockSpec((_NC, _BN, _DW), lambda i: (0, i, 0)),
        pl.BlockSpec((2, _BN, 128), lambda i: (0, i, 0)),
        pl.BlockSpec((_BN, 64), lambda i: (i, 0)),
        pl.BlockSpec((1, 1, _BN), lambda i: (i, 0, 0)),
        pl.BlockSpec((64, 1), lambda i: (0, 0)),
        pl.BlockSpec((1, 128), lambda i: (0, 0)),
        pl.BlockSpec((_G, 256), lambda i: (0, 0)),
        pl.BlockSpec((_G, 128), lambda i: (0, 0)),
        pl.BlockSpec((448, 256), lambda i: (0, 0)),
        pl.BlockSpec((1, 256), lambda i: (0, 0)),
        pl.BlockSpec((256, 128), lambda i: (0, 0)),
        pl.BlockSpec((1, 128), lambda i: (0, 0)),
        pl.BlockSpec((128, 32), lambda i: (0, 0)),
        pl.BlockSpec((1, 32), lambda i: (0, 0)),
    ],
    out_specs=pl.BlockSpec((_G, 32), lambda i: (0, 0)),
    out_shape=jax.ShapeDtypeStruct((_G, 32), jnp.float32),
    scratch_shapes=[pltpu.VMEM((_G, 64), jnp.float32)],
)


def kernel(x, edge_index, segment_ids, W1, W2, W3, wp1, bp1, wp2, bp2,
           wp3, bp3, Wd1, bd1, Wd2, bd2, Wd3, bd3):
    src = edge_index[0]
    dst = edge_index[1]
    seg3 = segment_ids.reshape(_N // _BN, 1, _BN)
    def chunked(a):
        a3 = a.reshape(1, _NCHUNK, _CH)
        return jnp.pad(a3, ((0, 0), (0, _NCHPAD - _NCHUNK), (0, 0)))

    src_lo = chunked(src)
    src_hi = chunked(src + _N)
    dst_c = chunked(dst)

    def b128(b):
        return jnp.broadcast_to(b.reshape(1, 1), (1, 128))

    z128 = jnp.zeros((_CH, 128), jnp.float32)
    deg2 = _deg_call(dst_c, jnp.ones((_CH, _DW), jnp.float32),
                     jnp.zeros((_ZR, _DW), jnp.float32))
    hw1, hn1 = _pre_call(deg2, x, W1)
    t1 = hn1.reshape(2 * _N, 128)
    agg1a = _scat_e128(t1, src_lo, dst_c, z128)
    agg1b = _scat_e128(t1, src_hi, dst_c, z128)
    g1, hw2, hn2 = _layer1(deg2, agg1a, agg1b, hw1, seg3, wp1, b128(bp1), W2)
    agg2 = _scat_e128(hn2, src_lo, dst_c, z128)
    g2, hw3, hn3 = _layer2(deg2, agg2, hw2, seg3, wp2, b128(bp2), W3)
    agg3 = _scat_e128(hn3, src_lo, dst_c, z128)
    out = _final_call(deg2, agg3, hw3, seg3, wp3, b128(bp3), g1, g2,
                      Wd1, bd1.reshape(1, 256), Wd2, bd2.reshape(1, 128),
                      Wd3, bd3.reshape(1, 32))
    return out
